# Initial kernel scaffold; baseline (speedup 1.0000x reference)
import jax
import jax.numpy as jnp
from jax.experimental import pallas as pl


def kernel(x_paper, x_author, edge_index_ap, edge_index_pa, W_ap_0, W_pa_0, W_sp_0, W_sa_0, W_ap_1, W_pa_1, W_sp_1, W_sa_1):
    raise NotImplementedError("write your pallas kernel here")


# speedup vs baseline: 5.3410x; 5.3410x over previous
import functools

import jax
import jax.numpy as jnp
from jax import lax
from jax.experimental import pallas as pl
from jax.experimental.pallas import tpu as pltpu
from jax.experimental.pallas import tpu_sc as plsc

N = 10000
D = 128
E = 160000
DAUG = 144
K = 125
NS = 16
EPT = E // NS
CH = EPT // K
ZR = 80
NZ = N // ZR
BM = 1000


def _aug_ones(bm):
    return (lax.broadcasted_iota(jnp.int32, (bm, DAUG - D), 1) == 0).astype(jnp.float32)


def _tc1_body(xp, xa, wsp, wpa, wsa, wap, sp, sa, tap, tpa):
    xpv = xp[...]
    xav = xa[...]
    sp[...] = jnp.dot(xpv, wsp[...], preferred_element_type=jnp.float32)
    sa[...] = jnp.dot(xav, wsa[...], preferred_element_type=jnp.float32)
    aug = _aug_ones(xpv.shape[0])
    tap[...] = jnp.concatenate(
        [jnp.dot(xav, wap[...], preferred_element_type=jnp.float32), aug], axis=1)
    tpa[...] = jnp.concatenate(
        [jnp.dot(xpv, wpa[...], preferred_element_type=jnp.float32), aug], axis=1)


def _tc2_body(aggp, agga, sp0, sa0, wsp, wpa, wsa, wap, sp1, sa1, tap, tpa):
    ap = aggp[...]
    aa = agga[...]
    hp = jax.nn.relu(sp0[...] + ap[:, :D] / jnp.clip(ap[:, D:D + 1], 1.0))
    ha = jax.nn.relu(sa0[...] + aa[:, :D] / jnp.clip(aa[:, D:D + 1], 1.0))
    sp1[...] = jnp.dot(hp, wsp[...], preferred_element_type=jnp.float32)
    sa1[...] = jnp.dot(ha, wsa[...], preferred_element_type=jnp.float32)
    aug = _aug_ones(hp.shape[0])
    tap[...] = jnp.concatenate(
        [jnp.dot(ha, wap[...], preferred_element_type=jnp.float32), aug], axis=1)
    tpa[...] = jnp.concatenate(
        [jnp.dot(hp, wpa[...], preferred_element_type=jnp.float32), aug], axis=1)


def _tc3_body(aggp, agga, sp1, sa1, zp, za):
    ap = aggp[...]
    aa = agga[...]
    zp[...] = sp1[...] + ap[:, :D] / jnp.clip(ap[:, D:D + 1], 1.0)
    za[...] = sa1[...] + aa[:, :D] / jnp.clip(aa[:, D:D + 1], 1.0)


_bs_x = pl.BlockSpec((BM, D), lambda i: (i, 0))
_bs_w = pl.BlockSpec((D, D), lambda i: (0, 0))
_bs_aug = pl.BlockSpec((BM, DAUG), lambda i: (i, 0))
_sds_x = jax.ShapeDtypeStruct((N, D), jnp.float32)
_sds_aug = jax.ShapeDtypeStruct((N, DAUG), jnp.float32)

_tc1 = pl.pallas_call(
    _tc1_body,
    grid=(N // BM,),
    in_specs=[_bs_x, _bs_x, _bs_w, _bs_w, _bs_w, _bs_w],
    out_specs=[_bs_x, _bs_x, _bs_aug, _bs_aug],
    out_shape=[_sds_x, _sds_x, _sds_aug, _sds_aug],
)

_tc2 = pl.pallas_call(
    _tc2_body,
    grid=(N // BM,),
    in_specs=[_bs_aug, _bs_aug, _bs_x, _bs_x, _bs_w, _bs_w, _bs_w, _bs_w],
    out_specs=[_bs_x, _bs_x, _bs_aug, _bs_aug],
    out_shape=[_sds_x, _sds_x, _sds_aug, _sds_aug],
)

_tc3 = pl.pallas_call(
    _tc3_body,
    grid=(N // BM,),
    in_specs=[_bs_aug, _bs_aug, _bs_x, _bs_x],
    out_specs=[_bs_x, _bs_x],
    out_shape=[_sds_x, _sds_x],
)


@functools.cache
def _make_sc_agg():
    mesh = plsc.VectorSubcoreMesh(core_axis_name="c", subcore_axis_name="s")
    return pl.kernel(
        _sc_agg_body,
        out_type=[jax.ShapeDtypeStruct((N, DAUG), jnp.float32),
                  jax.ShapeDtypeStruct((N, DAUG), jnp.float32)],
        mesh=mesh,
        scratch_types=[
            pltpu.VMEM((CH, K), jnp.int32),
            pltpu.VMEM((CH, K), jnp.int32),
            pltpu.VMEM((K, DAUG), jnp.float32),
            pltpu.VMEM_SHARED((N, DAUG), jnp.float32),
            pltpu.SemaphoreType.DMA,
        ],
        compiler_params=pltpu.CompilerParams(use_tc_tiling_on_sc=False),
    )


def _sc_agg_body(tap, tpa, src_ap, dst_ap, src_pa, dst_pa, zrows,
                 out_p, out_a, src_v, dst_v, rows_v, acc, sem):
    cid = lax.axis_index("c")
    sid = lax.axis_index("s")

    def run(table, src2d, dst2d, out):
        pltpu.sync_copy(src2d.at[pl.ds(sid * CH, CH)], src_v)
        pltpu.sync_copy(dst2d.at[pl.ds(sid * CH, CH)], dst_v)
        for k in range(pl.cdiv(NZ, NS)):
            j = sid + k * NS

            @pl.when(j < NZ)
            def _():
                pltpu.sync_copy(zrows, acc.at[pl.ds(j * ZR, ZR)])

        plsc.subcore_barrier()

        @pl.loop(0, CH)
        def _(j):
            pltpu.async_copy(table.at[src_v.at[j]], rows_v, sem).wait()
            pltpu.sync_copy(rows_v, acc.at[dst_v.at[j]], add=True)

        plsc.subcore_barrier()
        for k in range(pl.cdiv(NZ, NS)):
            j = sid + k * NS

            @pl.when(j < NZ)
            def _():
                pltpu.sync_copy(acc.at[pl.ds(j * ZR, ZR)],
                                out.at[pl.ds(j * ZR, ZR)])

    @pl.when(cid == 0)
    def _():
        run(tap, src_ap, dst_ap, out_p)

    @pl.when(cid == 1)
    def _():
        run(tpa, src_pa, dst_pa, out_a)


def kernel(x_paper, x_author, edge_index_ap, edge_index_pa,
           W_ap_0, W_pa_0, W_sp_0, W_sa_0,
           W_ap_1, W_pa_1, W_sp_1, W_sa_1):
    eap = edge_index_ap.astype(jnp.int32)
    epa = edge_index_pa.astype(jnp.int32)
    src_ap = eap[0].reshape(E // K, K)
    dst_ap = eap[1].reshape(E // K, K)
    src_pa = epa[0].reshape(E // K, K)
    dst_pa = epa[1].reshape(E // K, K)
    zrows = jnp.zeros((ZR, DAUG), jnp.float32)

    sc_agg = _make_sc_agg()
    sp0, sa0, tap0, tpa0 = _tc1(x_paper, x_author, W_sp_0, W_pa_0, W_sa_0, W_ap_0)
    aggp0, agga0 = sc_agg(tap0, tpa0, src_ap, dst_ap, src_pa, dst_pa, zrows)
    sp1, sa1, tap1, tpa1 = _tc2(aggp0, agga0, sp0, sa0,
                                W_sp_1, W_pa_1, W_sa_1, W_ap_1)
    aggp1, agga1 = sc_agg(tap1, tpa1, src_ap, dst_ap, src_pa, dst_pa, zrows)
    zp, za = _tc3(aggp1, agga1, sp1, sa1)
    return jnp.concatenate([zp, za], axis=0)
